# Initial kernel scaffold; baseline (speedup 1.0000x reference)
import jax
import jax.numpy as jnp
from jax.experimental import pallas as pl


def kernel(x, use_expert_i, conv_w, conv_b, gn_gamma, gn_beta):
    raise NotImplementedError("write your pallas kernel here")


# speedup vs baseline: 3.1432x; 3.1432x over previous
import jax
import jax.numpy as jnp
from jax.experimental import pallas as pl
from jax.experimental.pallas import tpu as pltpu

E = 8
C_IN = 256
C_OUT = 256
K = 5
G = 8
B = 64
L = 2048
EPS = 1e-5


def _body(idx_ref, x_ref, w_ref, p_ref, o_ref):
    x = x_ref[0]
    acc = jnp.zeros((C_OUT, L), dtype=jnp.float32)
    for k in range(K):
        acc += jax.lax.dot_general(
            w_ref[0, k], x[:, k:k + L],
            (((1,), (0,)), ((), ())),
            preferred_element_type=jnp.float32)
    acc += p_ref[0, 0].reshape(C_OUT, 1)
    yg = acc.reshape(G, (C_OUT // G) * L)
    mu = jnp.mean(yg, axis=1, keepdims=True)
    var = jnp.mean((yg - mu) * (yg - mu), axis=1, keepdims=True)
    yn = (yg - mu) * jax.lax.rsqrt(var + EPS)
    y = yn.reshape(C_OUT, L)
    y = y * p_ref[0, 1].reshape(C_OUT, 1) + p_ref[0, 2].reshape(C_OUT, 1)
    sp = jnp.logaddexp(y, 0.0)
    o_ref[0] = y * jnp.tanh(sp)


def kernel(x, use_expert_i, conv_w, conv_b, gn_gamma, gn_beta):
    xp = jnp.pad(x, ((0, 0), (0, 0), (K // 2, K // 2)))
    wt = jnp.transpose(conv_w, (0, 3, 1, 2))
    params = jnp.stack([conv_b, gn_gamma, gn_beta], axis=1)

    grid_spec = pltpu.PrefetchScalarGridSpec(
        num_scalar_prefetch=1,
        grid=(B,),
        in_specs=[
            pl.BlockSpec((1, C_IN, L + K - 1), lambda i, idx: (i, 0, 0)),
            pl.BlockSpec((1, K, C_OUT, C_IN), lambda i, idx: (idx[i], 0, 0, 0)),
            pl.BlockSpec((1, 3, C_OUT), lambda i, idx: (idx[i], 0, 0)),
        ],
        out_specs=pl.BlockSpec((1, C_OUT, L), lambda i, idx: (i, 0, 0)),
    )
    return pl.pallas_call(
        _body,
        grid_spec=grid_spec,
        out_shape=jax.ShapeDtypeStruct((B, C_OUT, L), jnp.float32),
        compiler_params=pltpu.CompilerParams(
            dimension_semantics=("arbitrary",),
        ),
    )(use_expert_i, xp, wt, params)


# speedup vs baseline: 3.6198x; 1.1516x over previous
import jax
import jax.numpy as jnp
from jax.experimental import pallas as pl
from jax.experimental.pallas import tpu as pltpu

E = 8
C_IN = 256
C_OUT = 256
K = 5
G = 8
B = 64
L = 2048
EPS = 1e-5


def _body(idx_ref, x_ref, w_ref, p_ref, o_ref):
    x = jnp.pad(x_ref[0].astype(jnp.bfloat16),
                ((0, 0), (K // 2, K // 2)))
    acc = jnp.zeros((C_OUT, L), dtype=jnp.float32)
    for k in range(K):
        acc += jax.lax.dot_general(
            w_ref[0, k], x[:, k:k + L],
            (((1,), (0,)), ((), ())),
            preferred_element_type=jnp.float32)
    acc += p_ref[0, 0].reshape(C_OUT, 1)
    yg = acc.reshape(G, (C_OUT // G) * L)
    mu = jnp.mean(yg, axis=1, keepdims=True)
    var = jnp.mean((yg - mu) * (yg - mu), axis=1, keepdims=True)
    yn = (yg - mu) * jax.lax.rsqrt(var + EPS)
    y = yn.reshape(C_OUT, L)
    y = y * p_ref[0, 1].reshape(C_OUT, 1) + p_ref[0, 2].reshape(C_OUT, 1)
    sp = jnp.logaddexp(y, 0.0)
    o_ref[0] = y * jnp.tanh(sp)


def kernel(x, use_expert_i, conv_w, conv_b, gn_gamma, gn_beta):
    wt = jnp.transpose(conv_w, (0, 3, 1, 2)).astype(jnp.bfloat16)
    params = jnp.stack([conv_b, gn_gamma, gn_beta], axis=1)

    grid_spec = pltpu.PrefetchScalarGridSpec(
        num_scalar_prefetch=1,
        grid=(B,),
        in_specs=[
            pl.BlockSpec((1, C_IN, L), lambda i, idx: (i, 0, 0)),
            pl.BlockSpec((1, K, C_OUT, C_IN), lambda i, idx: (idx[i], 0, 0, 0)),
            pl.BlockSpec((1, 3, C_OUT), lambda i, idx: (idx[i], 0, 0)),
        ],
        out_specs=pl.BlockSpec((1, C_OUT, L), lambda i, idx: (i, 0, 0)),
    )
    return pl.pallas_call(
        _body,
        grid_spec=grid_spec,
        out_shape=jax.ShapeDtypeStruct((B, C_OUT, L), jnp.float32),
        compiler_params=pltpu.CompilerParams(
            dimension_semantics=("arbitrary",),
        ),
    )(use_expert_i, x, wt, params)


# speedup vs baseline: 5.1785x; 1.4306x over previous
import jax
import jax.numpy as jnp
from jax.experimental import pallas as pl
from jax.experimental.pallas import tpu as pltpu

E = 8
C_IN = 256
C_OUT = 256
K = 5
G = 8
B = 64
L = 2048
EPS = 1e-5


def _body(idx_ref, x_ref, w_ref, p_ref, o_ref):
    xp = jnp.pad(x_ref[0].astype(jnp.bfloat16),
                 ((0, 0), (K // 2, K // 2)))
    xs = jnp.concatenate([xp[:, k:k + L] for k in range(K)], axis=0)
    acc = jax.lax.dot_general(
        w_ref[0], xs, (((1,), (0,)), ((), ())),
        preferred_element_type=jnp.float32)
    acc += p_ref[0, 0].reshape(C_OUT, 1)
    cpg = C_OUT // G
    n = cpg * L
    s1 = jnp.sum(acc, axis=1, keepdims=True)
    s2 = jnp.sum(acc * acc, axis=1, keepdims=True)
    gi = jax.lax.broadcasted_iota(jnp.int32, (C_OUT, C_OUT), 0) // cpg
    gj = jax.lax.broadcasted_iota(jnp.int32, (C_OUT, C_OUT), 1) // cpg
    gmask = (gi == gj).astype(jnp.float32)
    gs = jax.lax.dot_general(
        gmask, jnp.concatenate([s1, s2], axis=1),
        (((1,), (0,)), ((), ())),
        preferred_element_type=jnp.float32,
        precision=jax.lax.Precision.HIGHEST)
    mu_c = gs[:, 0:1] / n
    var_c = gs[:, 1:2] / n - mu_c * mu_c
    r_c = jax.lax.rsqrt(var_c + EPS)
    scale = r_c * p_ref[0, 1].reshape(C_OUT, 1)
    shift = p_ref[0, 2].reshape(C_OUT, 1) - mu_c * scale
    y = acc * scale + shift
    sp = jnp.logaddexp(y, 0.0)
    o_ref[0] = y * jnp.tanh(sp)


def kernel(x, use_expert_i, conv_w, conv_b, gn_gamma, gn_beta):
    wt = (jnp.transpose(conv_w, (0, 1, 3, 2))
          .reshape(E, C_OUT, K * C_IN).astype(jnp.bfloat16))
    params = jnp.stack([conv_b, gn_gamma, gn_beta], axis=1)

    grid_spec = pltpu.PrefetchScalarGridSpec(
        num_scalar_prefetch=1,
        grid=(B,),
        in_specs=[
            pl.BlockSpec((1, C_IN, L), lambda i, idx: (i, 0, 0)),
            pl.BlockSpec((1, C_OUT, K * C_IN), lambda i, idx: (idx[i], 0, 0)),
            pl.BlockSpec((1, 3, C_OUT), lambda i, idx: (idx[i], 0, 0)),
        ],
        out_specs=pl.BlockSpec((1, C_OUT, L), lambda i, idx: (i, 0, 0)),
    )
    return pl.pallas_call(
        _body,
        grid_spec=grid_spec,
        out_shape=jax.ShapeDtypeStruct((B, C_OUT, L), jnp.float32),
        compiler_params=pltpu.CompilerParams(
            dimension_semantics=("arbitrary",),
        ),
    )(use_expert_i, x, wt, params)


# speedup vs baseline: 5.9584x; 1.1506x over previous
import jax
import jax.numpy as jnp
from jax.experimental import pallas as pl
from jax.experimental.pallas import tpu as pltpu

E = 8
C_IN = 256
C_OUT = 256
K = 5
G = 8
B = 64
L = 2048
EPS = 1e-5


def _body(idx_ref, x_ref, w_ref, p_ref, m_ref, o_ref):
    xp = jnp.pad(x_ref[0].astype(jnp.bfloat16),
                 ((0, 0), (K // 2, K // 2)))
    xs = jnp.concatenate([xp[:, k:k + L] for k in range(K)], axis=0)
    acc = jax.lax.dot_general(
        w_ref[0], xs, (((1,), (0,)), ((), ())),
        preferred_element_type=jnp.float32)
    acc += p_ref[0, 0].reshape(C_OUT, 1)
    cpg = C_OUT // G
    n = cpg * L
    s1 = jnp.sum(acc, axis=1, keepdims=True)
    s2 = jnp.sum(acc * acc, axis=1, keepdims=True)
    gs = jax.lax.dot_general(
        m_ref[...], jnp.concatenate([s1, s2], axis=1),
        (((1,), (0,)), ((), ())),
        preferred_element_type=jnp.float32,
        precision=jax.lax.Precision.HIGHEST)
    mu_c = gs[:, 0:1] / n
    var_c = gs[:, 1:2] / n - mu_c * mu_c
    r_c = jax.lax.rsqrt(var_c + EPS)
    scale = r_c * p_ref[0, 1].reshape(C_OUT, 1)
    shift = p_ref[0, 2].reshape(C_OUT, 1) - mu_c * scale
    y = acc * scale + shift
    u = jnp.exp(jnp.minimum(y, 30.0))
    num = u * (u + 2.0)
    o_ref[0] = y * (num / (num + 2.0))


def kernel(x, use_expert_i, conv_w, conv_b, gn_gamma, gn_beta):
    wt = (jnp.transpose(conv_w, (0, 1, 3, 2))
          .reshape(E, C_OUT, K * C_IN).astype(jnp.bfloat16))
    params = jnp.stack([conv_b, gn_gamma, gn_beta], axis=1)
    cpg = C_OUT // G
    gi = jnp.arange(C_OUT, dtype=jnp.int32) // cpg
    gmask = (gi[:, None] == gi[None, :]).astype(jnp.float32)

    grid_spec = pltpu.PrefetchScalarGridSpec(
        num_scalar_prefetch=1,
        grid=(B,),
        in_specs=[
            pl.BlockSpec((1, C_IN, L), lambda i, idx: (i, 0, 0)),
            pl.BlockSpec((1, C_OUT, K * C_IN), lambda i, idx: (idx[i], 0, 0)),
            pl.BlockSpec((1, 3, C_OUT), lambda i, idx: (idx[i], 0, 0)),
            pl.BlockSpec((C_OUT, C_OUT), lambda i, idx: (0, 0)),
        ],
        out_specs=pl.BlockSpec((1, C_OUT, L), lambda i, idx: (i, 0, 0)),
    )
    return pl.pallas_call(
        _body,
        grid_spec=grid_spec,
        out_shape=jax.ShapeDtypeStruct((B, C_OUT, L), jnp.float32),
        compiler_params=pltpu.CompilerParams(
            dimension_semantics=("arbitrary",),
        ),
    )(use_expert_i, x, wt, params, gmask)
